# baseline (speedup 1.0000x reference)
import functools

import jax
import jax.numpy as jnp
from jax import lax
from jax.experimental import pallas as pl
from jax.experimental.pallas import tpu as pltpu
from jax.experimental.pallas import tpu_sc as plsc

N = 10000
NP = 10240
D = 128
NC = 2
NS = 16
NW = NC * NS
C = 128
RPT = NP // NS

_mesh = plsc.VectorSubcoreMesh(core_axis_name="c", subcore_axis_name="s")
_sc_params = pltpu.CompilerParams(needs_layout_passes=False)


def _zero_acc_slice(zbuf, acc, s, width):
    @pl.loop(0, 128)
    def _(r):
        for jj in range(width // 16):
            zbuf[r, pl.ds(jj * 16, 16)] = jnp.zeros((16,), jnp.float32)

    for t in range(RPT // 128):
        pltpu.sync_copy(zbuf, acc.at[pl.ds(s * RPT + t * 128, 128)])


def _make_deg_kernel(k2):
    @functools.partial(
        pl.kernel,
        out_type=jax.ShapeDtypeStruct((NC, NP), jnp.float32),
        mesh=_mesh,
        compiler_params=_sc_params,
        scratch_types=[
            pltpu.VMEM((12, C), jnp.int32),
            pltpu.VMEM((12, C), jnp.float32),
            pltpu.VMEM((RPT,), jnp.float32),
            pltpu.VMEM_SHARED((NP,), jnp.float32),
            pltpu.SemaphoreType.DMA,
            pltpu.SemaphoreType.DMA,
        ],
    )
    def deg_kernel(dst_hbm, ew_hbm, out_hbm, dstv, ewv, zv, dacc, isem, ssem):
        c = lax.axis_index("c")
        s = lax.axis_index("s")
        wid = s * NC + c
        base = wid * k2 * C

        @pl.loop(0, RPT // 16)
        def _(r):
            zv[pl.ds(r * 16, 16)] = jnp.zeros((16,), jnp.float32)

        pltpu.sync_copy(zv, dacc.at[pl.ds(s * RPT, RPT)])
        plsc.subcore_barrier()

        bd = 12

        @pl.loop(0, k2 // bd)
        def _(t):
            j = base + bd * t * C
            idx_d = []
            for q in range(bd):
                off = j + q * C
                idx_d += [
                    pltpu.async_copy(dst_hbm.at[pl.ds(off, C)], dstv.at[q], isem),
                    pltpu.async_copy(ew_hbm.at[pl.ds(off, C)], ewv.at[q], isem),
                ]
            for d in idx_d:
                d.wait()
            sc_d = [pltpu.async_copy(ewv.at[q], dacc.at[dstv.at[q]], ssem,
                                     add=True) for q in range(bd)]
            for d in sc_d:
                d.wait()

        plsc.subcore_barrier()
        pltpu.sync_copy(dacc.at[pl.ds(s * RPT, RPT)],
                        out_hbm.at[c, pl.ds(s * RPT, RPT)])

    return deg_kernel


def _make_agg_kernel(k2):
    bd = 12
    qn = k2 // bd

    @functools.partial(
        pl.kernel,
        out_type=jax.ShapeDtypeStruct((NC, NP, D), jnp.float32),
        mesh=_mesh,
        compiler_params=_sc_params,
        scratch_types=[
            pltpu.VMEM((12, C), jnp.int32),
            pltpu.VMEM((12, C), jnp.int32),
            pltpu.VMEM((12, C), jnp.float32),
            pltpu.VMEM((C, D), jnp.float32),
            pltpu.VMEM((C, D), jnp.float32),
            pltpu.VMEM_SHARED((NP, D), jnp.float32),
            pltpu.SemaphoreType.DMA, pltpu.SemaphoreType.DMA,
            pltpu.SemaphoreType.DMA, pltpu.SemaphoreType.DMA,
            pltpu.SemaphoreType.DMA,
        ],
    )
    def agg_kernel(h_hbm, src_hbm, dst_hbm, ew_hbm, out_hbm,
                   srcv, dstv, ewv, rows0, rows1,
                   acc, g0, g1, s0, s1, isem):
        c = lax.axis_index("c")
        s = lax.axis_index("s")
        wid = s * NC + c
        base = wid * k2 * C

        _zero_acc_slice(rows1, acc, s, D)
        plsc.subcore_barrier()

        def scale(rows, q):
            @pl.loop(0, C, unroll=8)
            def _(e):
                bc = plsc.load_gather(
                    ewv, [jnp.full((16,), q, jnp.int32),
                          jnp.full((16,), e, jnp.int32)])
                for jj in range(D // 16):
                    sl = pl.ds(jj * 16, 16)
                    rows[e, sl] = rows[e, sl] * bc

        @pl.loop(0, qn)
        def _(t):
            j = base + bd * t * C
            idx_d = []
            for q in range(bd):
                off = j + q * C
                idx_d += [
                    pltpu.async_copy(src_hbm.at[pl.ds(off, C)], srcv.at[q], isem),
                    pltpu.async_copy(dst_hbm.at[pl.ds(off, C)], dstv.at[q], isem),
                    pltpu.async_copy(ew_hbm.at[pl.ds(off, C)], ewv.at[q], isem),
                ]
            rows = [rows0, rows1]
            gsem = [g0, g1]
            ssem = [s0, s1]
            gd = [None, None]
            sd = [None, None]
            for d in idx_d[:3]:
                d.wait()
            gd[0] = pltpu.async_copy(h_hbm.at[srcv.at[0]], rows0, g0)
            for d in idx_d[3:6]:
                d.wait()
            gd[1] = pltpu.async_copy(h_hbm.at[srcv.at[1]], rows1, g1)
            for d in idx_d[6:]:
                d.wait()
            for q in range(bd):
                p = q % 2
                gd[p].wait()
                scale(rows[p], q)
                sd[p] = pltpu.async_copy(rows[p], acc.at[dstv.at[q]],
                                         ssem[p], add=True)
                if q >= 1 and q + 1 < bd:
                    pp = (q - 1) % 2
                    sd[pp].wait()
                    gd[pp] = pltpu.async_copy(
                        h_hbm.at[srcv.at[q + 1]], rows[pp], gsem[pp])
            sd[bd % 2].wait()
            sd[(bd - 1) % 2].wait()

        plsc.subcore_barrier()

        out_d = []
        for t in range(RPT // 128):
            rr = s * RPT + t * 128
            out_d.append(pltpu.async_copy(
                acc.at[pl.ds(rr, 128)], out_hbm.at[c, pl.ds(rr, 128)], g0))
        for d in out_d:
            d.wait()

    return agg_kernel


def _mm1s_body(x_ref, w_ref, degp_ref, dis_ref, hp_ref):
    d = degp_ref[0] + degp_ref[1]
    dis = jnp.where(d > 0, lax.rsqrt(d), 0.0)
    dis_ref[...] = dis
    hp_ref[...] = dis * lax.dot_general(
        x_ref[...], w_ref[...], (((1,), (1,)), ((), ())),
        preferred_element_type=jnp.float32)


def _mm1s(x_pad, w1, degp_col):
    return pl.pallas_call(
        _mm1s_body,
        grid=(NP // 512,),
        in_specs=[
            pl.BlockSpec((512, D), lambda i: (i, 0)),
            pl.BlockSpec((D, D), lambda i: (0, 0)),
            pl.BlockSpec((NC, 512, 1), lambda i: (0, i, 0)),
        ],
        out_specs=[
            pl.BlockSpec((512, 1), lambda i: (i, 0)),
            pl.BlockSpec((512, D), lambda i: (i, 0)),
        ],
        out_shape=[
            jax.ShapeDtypeStruct((NP, 1), jnp.float32),
            jax.ShapeDtypeStruct((NP, D), jnp.float32),
        ],
    )(x_pad, w1, degp_col)


def _bn_stats(z):
    ri = lax.broadcasted_iota(jnp.int32, (NP, D), 0)
    msk = ri < N
    zm = jnp.where(msk, z, 0.0)
    mean = jnp.sum(zm, axis=0, keepdims=True) * (1.0 / N)
    zc = jnp.where(msk, z - mean, 0.0)
    var = jnp.sum(zc * zc, axis=0, keepdims=True) * (1.0 / N)
    return mean, var


def _bn1mm2_body(p_ref, dis_ref, g_ref, be_ref, w2_ref, h2_ref):
    z = dis_ref[...] * (p_ref[0] + p_ref[1])
    mean, var = _bn_stats(z)
    zn = g_ref[...] * (z - mean) * lax.rsqrt(var + 1e-5) + be_ref[...]
    zr = jnp.maximum(zn, 0.0)
    h2_ref[...] = dis_ref[...] * lax.dot_general(
        zr, w2_ref[...], (((1,), (1,)), ((), ())),
        preferred_element_type=jnp.float32)


def _bn2_body(p_ref, dis_ref, x_ref, g_ref, be_ref, o_ref):
    z = dis_ref[...] * (p_ref[0] + p_ref[1])
    mean, var = _bn_stats(z)
    zn = g_ref[...] * (z - mean) * lax.rsqrt(var + 1e-5) + be_ref[...]
    o_ref[...] = jnp.maximum(zn + x_ref[...], 0.0)


def kernel(x, edge_index, edge_weight, W1, b1, g1, be1, W2, b2, g2, be2):
    del b1, b2
    e = edge_index.shape[1]
    et = e + N
    blk = NW * C * 4
    ep = ((et + blk - 1) // blk) * blk
    pad = ep - et
    k2 = ep // (NW * C)

    loops = jnp.arange(N, dtype=jnp.int32)
    pad_i = (jnp.arange(pad, dtype=jnp.int32) * 7) % N
    src1 = jnp.concatenate([edge_index[0], loops, pad_i])
    dst1 = jnp.concatenate([edge_index[1], loops, pad_i])
    ew1 = jnp.concatenate([
        edge_weight, jnp.ones((N,), jnp.float32), jnp.zeros((pad,), jnp.float32)])
    x_pad = jnp.pad(x, ((0, NP - N), (0, 0)))

    degp = _make_deg_kernel(k2)(dst1, ew1)
    dis_col, h1p = _mm1s(x_pad, W1, degp.reshape(NC, NP, 1))

    agg = _make_agg_kernel(k2)
    p1 = agg(h1p, src1, dst1, ew1)

    h2p = pl.pallas_call(
        _bn1mm2_body,
        out_shape=jax.ShapeDtypeStruct((NP, D), jnp.float32),
    )(p1, dis_col, g1.reshape(1, D), be1.reshape(1, D), W2)

    p2 = agg(h2p, src1, dst1, ew1)

    out_pad = pl.pallas_call(
        _bn2_body,
        out_shape=jax.ShapeDtypeStruct((NP, D), jnp.float32),
    )(p2, dis_col, x_pad, g2.reshape(1, D), be2.reshape(1, D))
    return out_pad[:N]


# speedup vs baseline: 1.1369x; 1.1369x over previous
import functools

import jax
import jax.numpy as jnp
from jax import lax
from jax.experimental import pallas as pl
from jax.experimental.pallas import tpu as pltpu
from jax.experimental.pallas import tpu_sc as plsc

N = 10000
NP = 10240
D = 128
NC = 2
NS = 16
NW = NC * NS
C = 128
RPT = NP // NS

_mesh = plsc.VectorSubcoreMesh(core_axis_name="c", subcore_axis_name="s")
_sc_params = pltpu.CompilerParams(needs_layout_passes=False)


def _zero_acc_slice(zbuf, acc, s, width):
    @pl.loop(0, 128)
    def _(r):
        for jj in range(width // 16):
            zbuf[r, pl.ds(jj * 16, 16)] = jnp.zeros((16,), jnp.float32)

    for t in range(RPT // 128):
        pltpu.sync_copy(zbuf, acc.at[pl.ds(s * RPT + t * 128, 128)])


def _make_deg_kernel(k2):
    @functools.partial(
        pl.kernel,
        out_type=jax.ShapeDtypeStruct((NC, NP), jnp.float32),
        mesh=_mesh,
        compiler_params=_sc_params,
        scratch_types=[
            pltpu.VMEM((12, C), jnp.int32),
            pltpu.VMEM((12, C), jnp.float32),
            pltpu.VMEM((RPT,), jnp.float32),
            pltpu.VMEM_SHARED((NP,), jnp.float32),
            pltpu.SemaphoreType.DMA,
            pltpu.SemaphoreType.DMA,
        ],
    )
    def deg_kernel(dst_hbm, ew_hbm, out_hbm, dstv, ewv, zv, dacc, isem, ssem):
        c = lax.axis_index("c")
        s = lax.axis_index("s")
        wid = s * NC + c
        base = wid * k2 * C

        @pl.loop(0, RPT // 16)
        def _(r):
            zv[pl.ds(r * 16, 16)] = jnp.zeros((16,), jnp.float32)

        pltpu.sync_copy(zv, dacc.at[pl.ds(s * RPT, RPT)])
        plsc.subcore_barrier()

        bd = 12

        @pl.loop(0, k2 // bd)
        def _(t):
            j = base + bd * t * C
            idx_d = []
            for q in range(bd):
                off = j + q * C
                idx_d += [
                    pltpu.async_copy(dst_hbm.at[pl.ds(off, C)], dstv.at[q], isem),
                    pltpu.async_copy(ew_hbm.at[pl.ds(off, C)], ewv.at[q], isem),
                ]
            for d in idx_d:
                d.wait()
            sc_d = [pltpu.async_copy(ewv.at[q], dacc.at[dstv.at[q]], ssem,
                                     add=True) for q in range(bd)]
            for d in sc_d:
                d.wait()

        plsc.subcore_barrier()
        pltpu.sync_copy(dacc.at[pl.ds(s * RPT, RPT)],
                        out_hbm.at[c, pl.ds(s * RPT, RPT)])

    return deg_kernel


def _make_agg_kernel(k2):
    qn = k2 // 6

    @functools.partial(
        pl.kernel,
        out_type=jax.ShapeDtypeStruct((NC, NP, D), jnp.float32),
        mesh=_mesh,
        compiler_params=_sc_params,
        scratch_types=[
            pltpu.VMEM((6, C), jnp.int32),
            pltpu.VMEM((6, C), jnp.int32),
            pltpu.VMEM((6, C), jnp.float32),
            pltpu.VMEM((C, D), jnp.float32),
            pltpu.VMEM((C, D), jnp.float32),
            pltpu.VMEM_SHARED((NP, D), jnp.float32),
            pltpu.SemaphoreType.DMA, pltpu.SemaphoreType.DMA,
            pltpu.SemaphoreType.DMA, pltpu.SemaphoreType.DMA,
            pltpu.SemaphoreType.DMA,
        ],
    )
    def agg_kernel(h_hbm, src_hbm, dst_hbm, ew_hbm, out_hbm,
                   srcv, dstv, ewv, rows0, rows1,
                   acc, g0, g1, s0, s1, isem):
        c = lax.axis_index("c")
        s = lax.axis_index("s")
        wid = s * NC + c
        base = wid * k2 * C

        _zero_acc_slice(rows1, acc, s, D)
        plsc.subcore_barrier()

        def scale(rows, q):
            @pl.loop(0, C, unroll=8)
            def _(e):
                bc = plsc.load_gather(
                    ewv, [jnp.full((16,), q, jnp.int32),
                          jnp.full((16,), e, jnp.int32)])
                for jj in range(D // 16):
                    sl = pl.ds(jj * 16, 16)
                    rows[e, sl] = rows[e, sl] * bc

        @pl.loop(0, qn)
        def _(t):
            j = base + 6 * t * C
            idx_d = []
            for q in range(6):
                off = j + q * C
                idx_d += [
                    pltpu.async_copy(src_hbm.at[pl.ds(off, C)], srcv.at[q], isem),
                    pltpu.async_copy(dst_hbm.at[pl.ds(off, C)], dstv.at[q], isem),
                    pltpu.async_copy(ew_hbm.at[pl.ds(off, C)], ewv.at[q], isem),
                ]
            for d in idx_d[:3]:
                d.wait()
            g0d = pltpu.async_copy(h_hbm.at[srcv.at[0]], rows0, g0)
            for d in idx_d[3:6]:
                d.wait()
            g1d = pltpu.async_copy(h_hbm.at[srcv.at[1]], rows1, g1)
            for d in idx_d[6:]:
                d.wait()
            g0d.wait()
            scale(rows0, 0)
            s0d = pltpu.async_copy(rows0, acc.at[dstv.at[0]], s0, add=True)
            g1d.wait()
            scale(rows1, 1)
            s1d = pltpu.async_copy(rows1, acc.at[dstv.at[1]], s1, add=True)
            s0d.wait()
            g2d = pltpu.async_copy(h_hbm.at[srcv.at[2]], rows0, g0)
            s1d.wait()
            g3d = pltpu.async_copy(h_hbm.at[srcv.at[3]], rows1, g1)
            g2d.wait()
            scale(rows0, 2)
            s2d = pltpu.async_copy(rows0, acc.at[dstv.at[2]], s0, add=True)
            g3d.wait()
            scale(rows1, 3)
            s3d = pltpu.async_copy(rows1, acc.at[dstv.at[3]], s1, add=True)
            s2d.wait()
            g4d = pltpu.async_copy(h_hbm.at[srcv.at[4]], rows0, g0)
            s3d.wait()
            g5d = pltpu.async_copy(h_hbm.at[srcv.at[5]], rows1, g1)
            g4d.wait()
            scale(rows0, 4)
            s4d = pltpu.async_copy(rows0, acc.at[dstv.at[4]], s0, add=True)
            g5d.wait()
            scale(rows1, 5)
            s5d = pltpu.async_copy(rows1, acc.at[dstv.at[5]], s1, add=True)
            s4d.wait()
            s5d.wait()

        plsc.subcore_barrier()

        out_d = []
        for t in range(RPT // 128):
            rr = s * RPT + t * 128
            out_d.append(pltpu.async_copy(
                acc.at[pl.ds(rr, 128)], out_hbm.at[c, pl.ds(rr, 128)], g0))
        for d in out_d:
            d.wait()

    return agg_kernel


def _mm1s_body(x_ref, w_ref, degp_ref, dis_ref, hp_ref):
    d = degp_ref[0] + degp_ref[1]
    dis = jnp.where(d > 0, lax.rsqrt(d), 0.0)
    dis_ref[...] = dis
    hp_ref[...] = dis * lax.dot_general(
        x_ref[...], w_ref[...], (((1,), (1,)), ((), ())),
        preferred_element_type=jnp.float32)


def _mm1s(x_pad, w1, degp_col):
    return pl.pallas_call(
        _mm1s_body,
        grid=(NP // 512,),
        in_specs=[
            pl.BlockSpec((512, D), lambda i: (i, 0)),
            pl.BlockSpec((D, D), lambda i: (0, 0)),
            pl.BlockSpec((NC, 512, 1), lambda i: (0, i, 0)),
        ],
        out_specs=[
            pl.BlockSpec((512, 1), lambda i: (i, 0)),
            pl.BlockSpec((512, D), lambda i: (i, 0)),
        ],
        out_shape=[
            jax.ShapeDtypeStruct((NP, 1), jnp.float32),
            jax.ShapeDtypeStruct((NP, D), jnp.float32),
        ],
    )(x_pad, w1, degp_col)


def _bn_stats(z):
    ri = lax.broadcasted_iota(jnp.int32, (NP, D), 0)
    msk = ri < N
    zm = jnp.where(msk, z, 0.0)
    mean = jnp.sum(zm, axis=0, keepdims=True) * (1.0 / N)
    zc = jnp.where(msk, z - mean, 0.0)
    var = jnp.sum(zc * zc, axis=0, keepdims=True) * (1.0 / N)
    return mean, var


def _bn1mm2_body(p_ref, dis_ref, g_ref, be_ref, w2_ref, h2_ref):
    z = dis_ref[...] * (p_ref[0] + p_ref[1])
    mean, var = _bn_stats(z)
    zn = g_ref[...] * (z - mean) * lax.rsqrt(var + 1e-5) + be_ref[...]
    zr = jnp.maximum(zn, 0.0)
    h2_ref[...] = dis_ref[...] * lax.dot_general(
        zr, w2_ref[...], (((1,), (1,)), ((), ())),
        preferred_element_type=jnp.float32)


def _bn2_body(p_ref, dis_ref, x_ref, g_ref, be_ref, o_ref):
    z = dis_ref[...] * (p_ref[0] + p_ref[1])
    mean, var = _bn_stats(z)
    zn = g_ref[...] * (z - mean) * lax.rsqrt(var + 1e-5) + be_ref[...]
    o_ref[...] = jnp.maximum(zn + x_ref[...], 0.0)


def kernel(x, edge_index, edge_weight, W1, b1, g1, be1, W2, b2, g2, be2):
    del b1, b2
    e = edge_index.shape[1]
    et = e + N
    blk = NW * C * 4
    ep = ((et + blk - 1) // blk) * blk
    pad = ep - et
    k2 = ep // (NW * C)

    loops = jnp.arange(N, dtype=jnp.int32)
    pad_i = (jnp.arange(pad, dtype=jnp.int32) * 7) % N
    src1 = jnp.concatenate([edge_index[0], loops, pad_i])
    dst1 = jnp.concatenate([edge_index[1], loops, pad_i])
    ew1 = jnp.concatenate([
        edge_weight, jnp.ones((N,), jnp.float32), jnp.zeros((pad,), jnp.float32)])
    x_pad = jnp.pad(x, ((0, NP - N), (0, 0)))

    degp = _make_deg_kernel(k2)(dst1, ew1)
    dis_col, h1p = _mm1s(x_pad, W1, degp.reshape(NC, NP, 1))

    agg = _make_agg_kernel(k2)
    p1 = agg(h1p, src1, dst1, ew1)

    h2p = pl.pallas_call(
        _bn1mm2_body,
        out_shape=jax.ShapeDtypeStruct((NP, D), jnp.float32),
    )(p1, dis_col, g1.reshape(1, D), be1.reshape(1, D), W2)

    p2 = agg(h2p, src1, dst1, ew1)

    out_pad = pl.pallas_call(
        _bn2_body,
        out_shape=jax.ShapeDtypeStruct((NP, D), jnp.float32),
    )(p2, dis_col, x_pad, g2.reshape(1, D), be2.reshape(1, D))
    return out_pad[:N]
